# Initial kernel scaffold; baseline (speedup 1.0000x reference)
import jax
import jax.numpy as jnp
from jax.experimental import pallas as pl


def kernel(positions, table):
    raise NotImplementedError("write your pallas kernel here")


# speedup vs baseline: 3.0523x; 3.0523x over previous
import functools

import jax
import jax.numpy as jnp
from jax import lax
from jax.experimental import pallas as pl
from jax.experimental.pallas import tpu as pltpu
from jax.experimental.pallas import tpu_sc as plsc

D = 64
NC, NS = 2, 16
NW = NC * NS
CHUNK = 512
SUB = 128


@functools.lru_cache(maxsize=None)
def _make_gather(B, V):
    b_per_w = B // NW
    nchunks = b_per_w // CHUNK
    mesh = plsc.VectorSubcoreMesh(core_axis_name="c", subcore_axis_name="s")

    @functools.partial(
        pl.kernel,
        mesh=mesh,
        out_type=jax.ShapeDtypeStruct((B, D), jnp.float32),
        scratch_types=[
            pltpu.VMEM((CHUNK,), jnp.int32),
            pltpu.VMEM((CHUNK, D), jnp.float32),
            pltpu.SemaphoreType.DMA,
        ],
        compiler_params=pltpu.CompilerParams(use_tc_tiling_on_sc=False),
    )
    def gather(table_hbm, pos_hbm, out_hbm, idx_v, rows_v, sem):
        wid = lax.axis_index("s") * NC + lax.axis_index("c")
        base = wid * b_per_w

        def body(c, carry):
            off = base + c * CHUNK
            pltpu.sync_copy(pos_hbm.at[pl.ds(off, CHUNK)], idx_v)
            cps = [
                pltpu.async_copy(
                    table_hbm.at[idx_v.at[pl.ds(j * SUB, SUB)]],
                    rows_v.at[pl.ds(j * SUB, SUB)],
                    sem,
                )
                for j in range(CHUNK // SUB)
            ]
            for cp in cps:
                cp.wait()
            pltpu.sync_copy(rows_v, out_hbm.at[pl.ds(off, CHUNK)])
            return carry

        lax.fori_loop(0, nchunks, body, 0)

    return gather


def kernel(positions, table):
    shp = positions.shape
    pos_flat = positions.reshape(-1).astype(jnp.int32)
    B = pos_flat.shape[0]
    out = _make_gather(B, table.shape[0])(table, pos_flat)
    return out.reshape(*shp, D)


# speedup vs baseline: 3.0528x; 1.0002x over previous
import functools

import jax
import jax.numpy as jnp
from jax import lax
from jax.experimental import pallas as pl
from jax.experimental.pallas import tpu as pltpu
from jax.experimental.pallas import tpu_sc as plsc

D = 64
NC, NS = 2, 16
NW = NC * NS
CHUNK = 512
SUB = 128
NSUB = CHUNK // SUB


@functools.lru_cache(maxsize=None)
def _make_gather(B, V):
    b_per_w = B // NW
    nchunks = b_per_w // CHUNK
    assert b_per_w % CHUNK == 0 and nchunks % 2 == 0 and nchunks >= 4
    mesh = plsc.VectorSubcoreMesh(core_axis_name="c", subcore_axis_name="s")

    @functools.partial(
        pl.kernel,
        mesh=mesh,
        out_type=jax.ShapeDtypeStruct((B, D), jnp.float32),
        scratch_types=[
            pltpu.VMEM((2, CHUNK), jnp.int32),
            pltpu.VMEM((2, CHUNK, D), jnp.float32),
            pltpu.SemaphoreType.DMA,
            pltpu.SemaphoreType.DMA,
            pltpu.SemaphoreType.DMA,
            pltpu.SemaphoreType.DMA,
            pltpu.SemaphoreType.DMA,
            pltpu.SemaphoreType.DMA,
        ],
        compiler_params=pltpu.CompilerParams(use_tc_tiling_on_sc=False),
    )
    def gather(table_hbm, pos_hbm, out_hbm, idx_v, rows_v, g0, g1, o0, o1, i0, i1):
        gsem = (g0, g1)
        osem = (o0, o1)
        isem = (i0, i1)
        wid = lax.axis_index("s") * NC + lax.axis_index("c")
        base = wid * b_per_w

        def idx_load(c, s):
            pltpu.async_copy(
                pos_hbm.at[pl.ds(base + c * CHUNK, CHUNK)], idx_v.at[s], isem[s]
            )

        def idx_wait(s):
            pltpu.make_async_copy(
                pos_hbm.at[pl.ds(0, CHUNK)], idx_v.at[s], isem[s]
            ).wait()

        def fire_gather(s):
            for j in range(NSUB):
                pltpu.async_copy(
                    table_hbm.at[idx_v.at[s, pl.ds(j * SUB, SUB)]],
                    rows_v.at[s, pl.ds(j * SUB, SUB)],
                    gsem[s],
                )

        def wait_gather(s):
            for j in range(NSUB):
                pltpu.make_async_copy(
                    table_hbm.at[idx_v.at[s, pl.ds(j * SUB, SUB)]],
                    rows_v.at[s, pl.ds(j * SUB, SUB)],
                    gsem[s],
                ).wait()

        def fire_out(c, s):
            pltpu.async_copy(
                rows_v.at[s], out_hbm.at[pl.ds(base + c * CHUNK, CHUNK)], osem[s]
            )

        def wait_out(s):
            pltpu.make_async_copy(
                rows_v.at[s], out_hbm.at[pl.ds(0, CHUNK)], osem[s]
            ).wait()

        idx_load(0, 0)
        idx_load(1, 1)
        idx_wait(0)
        fire_gather(0)
        idx_wait(1)
        fire_gather(1)

        def body(i, carry):
            g = 2 * i
            for s in (0, 1):
                c = g + s
                wait_gather(s)
                idx_load(c + 2, s)
                fire_out(c, s)
                wait_out(s)
                idx_wait(s)
                fire_gather(s)
            return carry

        lax.fori_loop(0, (nchunks - 2) // 2, body, 0)

        for s, c in ((0, nchunks - 2), (1, nchunks - 1)):
            wait_gather(s)
            fire_out(c, s)
        wait_out(0)
        wait_out(1)

    return gather


def kernel(positions, table):
    shp = positions.shape
    pos_flat = positions.reshape(-1).astype(jnp.int32)
    B = pos_flat.shape[0]
    out = _make_gather(B, table.shape[0])(table, pos_flat)
    return out.reshape(*shp, D)


# speedup vs baseline: 3.8660x; 1.2664x over previous
import functools

import jax
import jax.numpy as jnp
from jax import lax
from jax.experimental import pallas as pl
from jax.experimental.pallas import tpu as pltpu
from jax.experimental.pallas import tpu_sc as plsc

D = 64
L = 16
NC, NS = 2, 16
NW = NC * NS
CHUNK = 512
UNROLL = 16


@functools.lru_cache(maxsize=None)
def _make_gather(B, V):
    b_per_w = B // NW
    nchunks = b_per_w // CHUNK
    assert b_per_w % CHUNK == 0 and nchunks % 2 == 0 and nchunks >= 6
    mesh = plsc.VectorSubcoreMesh(core_axis_name="c", subcore_axis_name="s")

    @functools.partial(
        pl.kernel,
        mesh=mesh,
        out_type=jax.ShapeDtypeStruct((B * D,), jnp.float32),
        scratch_types=[
            pltpu.VMEM((V * D,), jnp.float32),
            pltpu.VMEM((2, CHUNK), jnp.int32),
            pltpu.VMEM((2, CHUNK * D), jnp.float32),
            pltpu.SemaphoreType.DMA,
            pltpu.SemaphoreType.DMA,
            pltpu.SemaphoreType.DMA,
            pltpu.SemaphoreType.DMA,
            pltpu.SemaphoreType.DMA,
        ],
        compiler_params=pltpu.CompilerParams(use_tc_tiling_on_sc=False),
    )
    def gather(table_hbm, pos_hbm, out_hbm, table_v, idx_v, rows_v,
               tsem, o0, o1, i0, i1):
        osem = (o0, o1)
        isem = (i0, i1)
        wid = lax.axis_index("s") * NC + lax.axis_index("c")
        base = wid * b_per_w

        def idx_load(c, s):
            pltpu.async_copy(
                pos_hbm.at[pl.ds(base + c * CHUNK, CHUNK)], idx_v.at[s], isem[s]
            )

        def idx_wait(s):
            pltpu.make_async_copy(
                pos_hbm.at[pl.ds(0, CHUNK)], idx_v.at[s], isem[s]
            ).wait()

        def fire_out(c, s):
            pltpu.async_copy(
                rows_v.at[s],
                out_hbm.at[pl.ds((base + c * CHUNK) * D, CHUNK * D)],
                osem[s],
            )

        def wait_out(s):
            pltpu.make_async_copy(
                rows_v.at[s], out_hbm.at[pl.ds(0, CHUNK * D)], osem[s]
            ).wait()

        def compute(s):
            def step(g, carry):
                r0 = g * UNROLL
                bases = idx_v[s, pl.ds(r0, UNROLL)] * D
                for k in range(UNROLL):
                    tb = bases[k]
                    rb = (r0 + k) * D
                    for q in range(0, D, L):
                        rows_v[s, pl.ds(rb + q, L)] = table_v[pl.ds(tb + q, L)]
                return carry

            lax.fori_loop(0, CHUNK // UNROLL, step, 0)

        pltpu.async_copy(table_hbm, table_v, tsem)
        pltpu.make_async_copy(table_hbm, table_v, tsem).wait()

        idx_load(0, 0)
        idx_load(1, 1)
        for c in (0, 1):
            s = c
            idx_wait(s)
            compute(s)
            fire_out(c, s)
            idx_load(c + 2, s)

        def body(i, carry):
            for s in (0, 1):
                c = 2 * i + s
                idx_wait(s)
                wait_out(s)
                compute(s)
                fire_out(c, s)
                idx_load(c + 2, s)
            return carry

        lax.fori_loop(1, nchunks // 2 - 1, body, 0)

        for c in (nchunks - 2, nchunks - 1):
            s = c % 2
            idx_wait(s)
            wait_out(s)
            compute(s)
            fire_out(c, s)
        wait_out(0)
        wait_out(1)

    return gather


def kernel(positions, table):
    shp = positions.shape
    pos_flat = positions.reshape(-1).astype(jnp.int32)
    B = pos_flat.shape[0]
    out = _make_gather(B, table.shape[0])(table.reshape(-1), pos_flat)
    return out.reshape(*shp, D)
